# baseline (speedup 1.0000x reference)
import functools

import jax
import jax.numpy as jnp
from jax.experimental import pallas as pl
from jax.experimental.pallas import tpu as pltpu

M = 4096
K = 128
OUT = 64
BM = 512
BN = 256
NP = M // BM
_F32 = jnp.float32
_PREC = jax.lax.Precision.DEFAULT
_NBLK = M // BN


def _step(c, xd, bd, xs_ref, bsr_ref, c0, n, cden, acc, s):
    cc = jnp.minimum(c0 + c, jnp.int32(_NBLK - 1))
    off = cc * jnp.int32(BN)
    xs = xs_ref[pl.ds(off, BN), :]
    bsb = bsr_ref[:, pl.ds(off, BN)]
    ss = jnp.sum(xs * xs, axis=1, keepdims=True)
    sst = jnp.transpose(ss)
    rs = jnp.where(sst < 1e-12, _F32(1e6), jax.lax.rsqrt(sst))
    p = jax.lax.dot_general(xd, xs, (((1,), (1,)), ((), ())),
                            precision=_PREC)
    q = jnp.maximum(p, 0.0) * rs
    mask = jnp.logical_and(bd == bsb, c < n)
    cm = jnp.where(mask, q, 0.0)
    w_inc = jnp.where(mask, q + cden, 0.0)
    acc = acc + jax.lax.dot_general(cm, xs, (((1,), (0,)), ((), ())),
                                    precision=_PREC)
    s = s + jnp.sum(w_inc, axis=1, keepdims=True)
    return acc, s


def _combine(xd, acc, s, w2):
    rinv = jnp.where(s > 0, 1.0 / s, 0.0)
    gx = acc * rinv
    dot = lambda a: jax.lax.dot_general(a, w2, (((1,), (1,)), ((), ())),
                                        precision=_PREC)
    num = dot(xd * gx)
    td = jnp.sqrt(dot(xd * xd) + 1e-6)
    gd = jnp.sqrt(dot(gx * gx) + 1e-6)
    return num / jnp.maximum(td * gd, 1e-6)


def _cross_kernel(win_ref, blc_ref, brc_ref, blr_ref, brr_ref,
                  xl_ref, xr_ref, w_ref, o1_ref, o2_ref):
    i = pl.program_id(0)
    offd = i * jnp.int32(BM)
    bdl = blc_ref[pl.ds(offd, BM), :]
    bdr = brc_ref[pl.ds(offd, BM), :]
    xdl = xl_ref[pl.ds(offd, BM), :]
    xdr = xr_ref[pl.ds(offd, BM), :]

    c0a = win_ref[0, i]
    na = win_ref[1, i]
    c0b = win_ref[2, i]
    nb = win_ref[3, i]
    dnl = jnp.sqrt(jnp.sum(xdl * xdl, axis=1, keepdims=True))
    dnr = jnp.sqrt(jnp.sum(xdr * xdr, axis=1, keepdims=True))
    cdl = _F32(1e-6) * dnl
    cdr = _F32(1e-6) * dnr

    _UNROLL = 3

    def body(c, carry):
        acc_a, s_a, acc_b, s_b = carry
        cu = c * jnp.int32(_UNROLL)
        for u in range(_UNROLL):
            ci = cu + jnp.int32(u)
            acc_a, s_a = _step(ci, xdl, bdl, xr_ref, brr_ref, c0a, na, cdl,
                               acc_a, s_a)
            acc_b, s_b = _step(ci, xdr, bdr, xl_ref, blr_ref, c0b, nb, cdr,
                               acc_b, s_b)
        return acc_a, s_a, acc_b, s_b

    z_acc = jnp.zeros((BM, K), _F32)
    z_s = jnp.zeros((BM, 1), _F32)
    n = jnp.maximum(na, nb)
    n2 = jax.lax.div(n + jnp.int32(_UNROLL - 1), jnp.int32(_UNROLL))
    acc_a, s_a, acc_b, s_b = jax.lax.fori_loop(
        jnp.int32(0), n2, body, (z_acc, z_s, z_acc, z_s))

    w2 = w_ref[...]
    w2 = w2 * w2
    o1_ref[...] = _combine(xdl, acc_a, s_a, w2)
    o2_ref[...] = _combine(xdr, acc_b, s_b, w2)


def _z():
    return jnp.int32(0)


@functools.partial(jax.jit, static_argnames=("interpret",))
def _run(x_left, batch_left, x_right, batch_right, weight, interpret=False):
    bl = batch_left.astype(jnp.int32)
    br = batch_right.astype(jnp.int32)
    blf = bl.astype(_F32)
    brf = br.astype(_F32)

    gl = bl.reshape(NP, BM)
    gr = br.reshape(NP, BM)
    cnt32 = lambda m: jnp.sum(m, axis=1, dtype=jnp.int32)
    sa = cnt32(br[None, :] < gl[:, :1])
    ea = cnt32(br[None, :] <= gl[:, -1:])
    sb = cnt32(bl[None, :] < gr[:, :1])
    eb = cnt32(bl[None, :] <= gr[:, -1:])
    c0a = sa // BN
    na = (ea + BN - 1) // BN - c0a
    c0b = sb // BN
    nb = (eb + BN - 1) // BN - c0b
    win = jnp.stack([c0a, na, c0b, nb]).astype(jnp.int32)

    full = lambda shape: pl.BlockSpec(shape, lambda i: (_z(), _z()))
    out_spec = pl.BlockSpec((BM, OUT), lambda i: (i, _z()))
    out1, out2 = pl.pallas_call(
        _cross_kernel,
        grid=(NP,),
        out_shape=[jax.ShapeDtypeStruct((M, OUT), _F32),
                   jax.ShapeDtypeStruct((M, OUT), _F32)],
        in_specs=[
            pl.BlockSpec((4, NP), lambda i: (_z(), _z()),
                         memory_space=pltpu.SMEM),
            full((M, 1)),
            full((M, 1)),
            full((1, M)),
            full((1, M)),
            full((M, K)),
            full((M, K)),
            full((OUT, K)),
        ],
        out_specs=[out_spec, out_spec],
        interpret=interpret,
    )(win, blf[:, None], brf[:, None], blf[None, :], brf[None, :],
      x_left, x_right, weight)
    return out1, out2


def kernel(x_left, batch_left, x_right, batch_right, weight):
    return _run(x_left, batch_left, x_right, batch_right, weight)


# speedup vs baseline: 1.3247x; 1.3247x over previous
import functools

import jax
import jax.numpy as jnp
from jax.experimental import pallas as pl
from jax.experimental.pallas import tpu as pltpu

M = 4096
K = 128
OUT = 64
BM = 512
BN = 256
NP = M // BM
_F32 = jnp.float32
_PREC = jax.lax.Precision.DEFAULT
_NBLK = M // BN


def _step(c, xd, bd, xs_ref, xsc_ref, bsr_ref, c0, n, cden, acc, s):
    cc = jnp.minimum(c0 + c, jnp.int32(_NBLK - 1))
    off = cc * jnp.int32(BN)
    xs = xs_ref[pl.ds(off, BN), :]
    xsc = xsc_ref[pl.ds(off, BN), :]
    bsb = bsr_ref[:, pl.ds(off, BN)]
    p = jax.lax.dot_general(xd, xsc, (((1,), (1,)), ((), ())),
                            precision=_PREC)
    q = jnp.maximum(p, 0.0)
    mask = jnp.logical_and(bd == bsb, c < n)
    cm = jnp.where(mask, q, 0.0)
    w_inc = jnp.where(mask, q + cden, 0.0)
    acc = acc + jax.lax.dot_general(cm, xs, (((1,), (0,)), ((), ())),
                                    precision=_PREC)
    s = s + jnp.sum(w_inc, axis=1, keepdims=True)
    return acc, s


def _combine(xd, acc, s, w2):
    rinv = jnp.where(s > 0, 1.0 / s, 0.0)
    gx = acc * rinv
    dot = lambda a: jax.lax.dot_general(a, w2, (((1,), (1,)), ((), ())),
                                        precision=_PREC)
    num = dot(xd * gx)
    td = jnp.sqrt(dot(xd * xd) + 1e-6)
    gd = jnp.sqrt(dot(gx * gx) + 1e-6)
    return num / jnp.maximum(td * gd, 1e-6)


def _cross_kernel(win_ref, blc_ref, brc_ref, blr_ref, brr_ref,
                  xl_ref, xr_ref, w_ref, o1_ref, o2_ref,
                  xlc_ref, xrc_ref):
    i = pl.program_id(0)

    @pl.when(i == 0)
    def _init():
        for src_ref, dst_ref in ((xl_ref, xlc_ref), (xr_ref, xrc_ref)):
            xs = src_ref[...]
            ssq = jnp.sum(xs * xs, axis=1, keepdims=True)
            rsn = jnp.where(ssq < 1e-12, _F32(1e6), jax.lax.rsqrt(ssq))
            dst_ref[...] = xs * rsn

    offd = i * jnp.int32(BM)
    bdl = blc_ref[pl.ds(offd, BM), :]
    bdr = brc_ref[pl.ds(offd, BM), :]
    xdl = xl_ref[pl.ds(offd, BM), :]
    xdr = xr_ref[pl.ds(offd, BM), :]

    c0a = win_ref[0, i]
    na = win_ref[1, i]
    c0b = win_ref[2, i]
    nb = win_ref[3, i]
    dnl = jnp.sqrt(jnp.sum(xdl * xdl, axis=1, keepdims=True))
    dnr = jnp.sqrt(jnp.sum(xdr * xdr, axis=1, keepdims=True))
    cdl = _F32(1e-6) * dnl
    cdr = _F32(1e-6) * dnr

    _UNROLL = 2

    def body(c, carry):
        acc_a, s_a, acc_b, s_b = carry
        cu = c * jnp.int32(_UNROLL)
        for u in range(_UNROLL):
            ci = cu + jnp.int32(u)
            acc_a, s_a = _step(ci, xdl, bdl, xr_ref, xrc_ref, brr_ref,
                               c0a, na, cdl, acc_a, s_a)
            acc_b, s_b = _step(ci, xdr, bdr, xl_ref, xlc_ref, blr_ref,
                               c0b, nb, cdr, acc_b, s_b)
        return acc_a, s_a, acc_b, s_b

    z_acc = jnp.zeros((BM, K), _F32)
    z_s = jnp.zeros((BM, 1), _F32)
    n = jnp.maximum(na, nb)
    n2 = jax.lax.div(n + jnp.int32(_UNROLL - 1), jnp.int32(_UNROLL))
    acc_a, s_a, acc_b, s_b = jax.lax.fori_loop(
        jnp.int32(0), n2, body, (z_acc, z_s, z_acc, z_s))

    w2 = w_ref[...]
    w2 = w2 * w2
    o1_ref[...] = _combine(xdl, acc_a, s_a, w2)
    o2_ref[...] = _combine(xdr, acc_b, s_b, w2)


def _z():
    return jnp.int32(0)


@functools.partial(jax.jit, static_argnames=("interpret",))
def _run(x_left, batch_left, x_right, batch_right, weight, interpret=False):
    bl = batch_left.astype(jnp.int32)
    br = batch_right.astype(jnp.int32)
    blf = bl.astype(_F32)
    brf = br.astype(_F32)

    gl = bl.reshape(NP, BM)
    gr = br.reshape(NP, BM)
    cnt32 = lambda m: jnp.sum(m, axis=1, dtype=jnp.int32)
    sa = cnt32(br[None, :] < gl[:, :1])
    ea = cnt32(br[None, :] <= gl[:, -1:])
    sb = cnt32(bl[None, :] < gr[:, :1])
    eb = cnt32(bl[None, :] <= gr[:, -1:])
    c0a = sa // BN
    na = (ea + BN - 1) // BN - c0a
    c0b = sb // BN
    nb = (eb + BN - 1) // BN - c0b
    win = jnp.stack([c0a, na, c0b, nb]).astype(jnp.int32)

    full = lambda shape: pl.BlockSpec(shape, lambda i: (_z(), _z()))
    out_spec = pl.BlockSpec((BM, OUT), lambda i: (i, _z()))
    out1, out2 = pl.pallas_call(
        _cross_kernel,
        grid=(NP,),
        out_shape=[jax.ShapeDtypeStruct((M, OUT), _F32),
                   jax.ShapeDtypeStruct((M, OUT), _F32)],
        in_specs=[
            pl.BlockSpec((4, NP), lambda i: (_z(), _z()),
                         memory_space=pltpu.SMEM),
            full((M, 1)),
            full((M, 1)),
            full((1, M)),
            full((1, M)),
            full((M, K)),
            full((M, K)),
            full((OUT, K)),
        ],
        out_specs=[out_spec, out_spec],
        scratch_shapes=[pltpu.VMEM((M, K), _F32), pltpu.VMEM((M, K), _F32)],
        interpret=interpret,
    )(win, blf[:, None], brf[:, None], blf[None, :], brf[None, :],
      x_left, x_right, weight)
    return out1, out2


def kernel(x_left, batch_left, x_right, batch_right, weight):
    return _run(x_left, batch_left, x_right, batch_right, weight)
